# baseline (speedup 1.0000x reference)
import dataclasses
import functools

import jax
import jax.numpy as jnp
from jax import lax
from jax.experimental import pallas as pl
from jax.experimental.pallas import tpu as pltpu
from jax.experimental.pallas import tpu_sc as plsc

B, N, E = 2, 10000, 160000
IMG, LOC, D, MID = 1024, 5, 128, 128
KG, TL, KGE, R, CONN = 1000, 8, 4000, 8, 4
VOCAB, NC = 10000, 3000

AW = 128
NP = 10240
CH = 128
EPS = E // 16
NCH = 80
ZR = 640
CROW = NP // CH

BN = 2000
KB = 1376
KEYS_PAD = 11008


def _img_proj_body(feat_ref, loc_ref, Wi_ref, bi_ref, Wl_ref, bl_ref,
                   Wr1_ref, Wr2_ref, br_ref,
                   x_ref, node0_ref, y1_ref, y2_ref):
    x = jax.lax.dot_general(
        feat_ref[...], Wi_ref[...], (((1,), (0,)), ((), ())),
        preferred_element_type=jnp.float32) + bi_ref[...]
    locp = jax.lax.dot_general(
        loc_ref[...], Wl_ref[...], (((1,), (0,)), ((), ())),
        preferred_element_type=jnp.float32) + bl_ref[...]
    x_ref[...] = x
    node0_ref[...] = x + locp
    y1_ref[...] = jax.lax.dot_general(
        x, Wr1_ref[...], (((1,), (0,)), ((), ())),
        preferred_element_type=jnp.float32)
    y2_ref[...] = jax.lax.dot_general(
        x, Wr2_ref[...], (((1,), (0,)), ((), ())),
        preferred_element_type=jnp.float32) + br_ref[...]


def _img_proj(feat, loc, Wi, bi, Wl, bl, Wr1, Wr2, br):
    nb = (B * N) // BN
    grid = (nb,)
    row = lambda i: (i, 0)
    rep = lambda i: (0, 0)
    out_sd = jax.ShapeDtypeStruct((B * N, D), jnp.float32)
    out_aw = jax.ShapeDtypeStruct((B * N, AW), jnp.float32)
    return pl.pallas_call(
        _img_proj_body,
        grid=grid,
        in_specs=[
            pl.BlockSpec((BN, IMG), row),
            pl.BlockSpec((BN, LOC), row),
            pl.BlockSpec((IMG, D), rep),
            pl.BlockSpec((1, D), rep),
            pl.BlockSpec((LOC, D), rep),
            pl.BlockSpec((1, D), rep),
            pl.BlockSpec((D, D), rep),
            pl.BlockSpec((D, D), rep),
            pl.BlockSpec((1, D), rep),
        ],
        out_specs=[pl.BlockSpec((BN, D), row), pl.BlockSpec((BN, D), row),
                   pl.BlockSpec((BN, AW), row), pl.BlockSpec((BN, D), row)],
        out_shape=[out_sd, out_sd, out_aw, out_sd],
        compiler_params=pltpu.CompilerParams(
            dimension_semantics=("arbitrary",)),
    )(feat, loc, Wi, bi, Wl, bl, Wr1, Wr2, br)


def _sc_params():
    cp = pltpu.CompilerParams()
    if "needs_layout_passes" in pltpu.CompilerParams.__dataclass_fields__:
        cp = dataclasses.replace(cp, needs_layout_passes=False)
    return cp


def _cnt_body(pk_hbm, cnt_hbm, i2_v, zbuf, lcnt, iidx, cnt_sh):
    c = lax.axis_index("c")
    s = lax.axis_index("s")

    @pl.loop(0, CROW)
    def _(i):
        for g in range(CH // 16):
            lcnt[i, pl.ds(g * 16, 16)] = jnp.zeros((16,), jnp.float32)
            zbuf[0, pl.ds(g * 16, 16)] = jnp.zeros((16,), jnp.float32)

    for g in range(CROW // 16):
        iidx[pl.ds(g * 16, 16)] = lax.iota(jnp.int32, 16) + g * 16

    @pl.when(s < CROW // 8)
    def _():
        for t in range(8):
            pltpu.sync_copy(zbuf, cnt_sh.at[pl.ds(s * 8 + t, 1)])

    pltpu.sync_copy(pk_hbm.at[c, s], i2_v)
    plsc.subcore_barrier()

    @pl.loop(0, NCH)
    def _(j):
        for g in range(CH // 16):
            v = i2_v[j, pl.ds(g * 16, 16)]
            plsc.addupdate_scatter(
                lcnt, [lax.shift_right_logical(v, 22),
                       lax.bitwise_and(lax.shift_right_logical(v, 15), 127)],
                jnp.ones((16,), jnp.float32))

    pltpu.sync_copy(lcnt, cnt_sh.at[iidx], add=True)
    plsc.subcore_barrier()

    @pl.when(s < CROW // 8)
    def _():
        pltpu.sync_copy(cnt_sh.at[pl.ds(s * 8, 8)],
                        cnt_hbm.at[c].at[pl.ds(s * 8, 8)])


def _edge_cnt(pk):
    mesh = plsc.VectorSubcoreMesh(core_axis_name="c", subcore_axis_name="s")
    return pl.kernel(
        _cnt_body,
        out_type=jax.ShapeDtypeStruct((B, CROW, CH), jnp.float32),
        mesh=mesh,
        scratch_types=[
            pltpu.VMEM((NCH, CH), jnp.int32),
            pltpu.VMEM((1, CH), jnp.float32),
            pltpu.VMEM((CROW, CH), jnp.float32),
            pltpu.VMEM((CROW,), jnp.int32),
            pltpu.VMEM_SHARED((CROW, CH), jnp.float32),
        ],
        compiler_params=_sc_params(),
    )(pk)


def _edge_seg_body(xa_hbm, pk_hbm, out_hbm, pk_v, i1a, i2a, i1b, i2b,
                   bufa, bufb, acc_sh, sema, semb):
    c = lax.axis_index("c")
    s = lax.axis_index("s")

    @pl.loop(0, CH)
    def _(i):
        for g in range(AW // 16):
            bufa[i, pl.ds(g * 16, 16)] = jnp.zeros((16,), jnp.float32)
    base = s * ZR
    for t in range(ZR // CH):
        pltpu.sync_copy(bufa, acc_sh.at[pl.ds(base + t * CH, CH)])

    pltpu.sync_copy(pk_hbm.at[c, s], pk_v)
    plsc.subcore_barrier()

    def _unpack(j, i1c, i2c):
        for g in range(CH // 16):
            v = pk_v[j, pl.ds(g * 16, 16)]
            i1c[0, pl.ds(g * 16, 16)] = lax.bitwise_and(v, 32767)
            i2c[0, pl.ds(g * 16, 16)] = lax.shift_right_logical(v, 15)

    @pl.loop(0, NCH, step=2)
    def _(j):
        _unpack(j, i1a, i2a)
        pltpu.async_copy(xa_hbm.at[i1a.at[0]], bufa, sema).wait()
        pltpu.sync_copy(bufa, acc_sh.at[i2a.at[0]], add=True)
        _unpack(j + 1, i1b, i2b)
        pltpu.async_copy(xa_hbm.at[i1b.at[0]], bufb, semb).wait()
        pltpu.sync_copy(bufb, acc_sh.at[i2b.at[0]], add=True)

    plsc.subcore_barrier()
    pltpu.sync_copy(acc_sh.at[pl.ds(base, ZR)],
                    out_hbm.at[c].at[pl.ds(base, ZR)])


def _edge_seg(xa, pk):
    mesh = plsc.VectorSubcoreMesh(core_axis_name="c", subcore_axis_name="s")
    return pl.kernel(
        _edge_seg_body,
        out_type=jax.ShapeDtypeStruct((B, NP, AW), jnp.float32),
        mesh=mesh,
        scratch_types=[
            pltpu.VMEM((NCH, CH), jnp.int32),
            pltpu.VMEM((1, CH), jnp.int32),
            pltpu.VMEM((1, CH), jnp.int32),
            pltpu.VMEM((1, CH), jnp.int32),
            pltpu.VMEM((1, CH), jnp.int32),
            pltpu.VMEM((CH, AW), jnp.float32),
            pltpu.VMEM((CH, AW), jnp.float32),
            pltpu.VMEM_SHARED((NP, AW), jnp.float32),
            pltpu.SemaphoreType.DMA,
            pltpu.SemaphoreType.DMA,
        ],
        compiler_params=_sc_params(),
    )(xa, pk)


def _attn_body(keys_ref, valid_ref, qp_ref, Wk_ref, v_ref, gvec_ref,
               Wre_ref, bre_ref, out_ref, num_ref, den_ref):
    k = pl.program_id(1)
    nb = pl.num_programs(1)

    @pl.when(k == 0)
    def _():
        num_ref[...] = jnp.zeros_like(num_ref)
        den_ref[...] = jnp.zeros_like(den_ref)

    keys = keys_ref[0]
    kp = jax.lax.dot_general(
        keys, Wk_ref[...], (((1,), (0,)), ((), ())),
        preferred_element_type=jnp.float32)
    qp = qp_ref[0]
    valid = valid_ref[0]
    v = v_ref[...]
    for r in range(R):
        t = jnp.tanh(kp + qp[r:r + 1, :])
        s = jnp.sum(t * v, axis=1, keepdims=True)
        es = jnp.where(valid.T > 0.0, jnp.exp(s), 0.0)
        num_ref[r:r + 1, :] += jax.lax.dot_general(
            es, keys, (((0,), (0,)), ((), ())),
            preferred_element_type=jnp.float32)
        den_ref[r:r + 1, :] += jnp.broadcast_to(jnp.sum(es), (1, D))

    @pl.when(k == nb - 1)
    def _():
        node_re = num_ref[...] / den_ref[...]
        pooled = jax.lax.dot_general(
            gvec_ref[0], node_re, (((1,), (0,)), ((), ())),
            preferred_element_type=jnp.float32)
        out_ref[0] = jax.lax.dot_general(
            pooled, Wre_ref[...], (((1,), (0,)), ((), ())),
            preferred_element_type=jnp.float32) + bre_ref[...]


def _attention(keys_p, valid_p, qp, Wk, v_att, gvec, Wre, bre):
    nb = KEYS_PAD // KB
    grid = (B, nb)
    return pl.pallas_call(
        _attn_body,
        grid=grid,
        in_specs=[
            pl.BlockSpec((1, KB, D), lambda b, k: (b, k, 0)),
            pl.BlockSpec((1, 1, KB), lambda b, k: (b * nb + k, 0, 0)),
            pl.BlockSpec((1, R, MID), lambda b, k: (b, 0, 0)),
            pl.BlockSpec((D, MID), lambda b, k: (0, 0)),
            pl.BlockSpec((1, MID), lambda b, k: (0, 0)),
            pl.BlockSpec((1, 1, R), lambda b, k: (b, 0, 0)),
            pl.BlockSpec((D, NC), lambda b, k: (0, 0)),
            pl.BlockSpec((1, NC), lambda b, k: (0, 0)),
        ],
        out_specs=pl.BlockSpec((1, 1, NC), lambda b, k: (b, 0, 0)),
        out_shape=jax.ShapeDtypeStruct((B, 1, NC), jnp.float32),
        scratch_shapes=[
            pltpu.VMEM((R, D), jnp.float32),
            pltpu.VMEM((R, D), jnp.float32),
        ],
        compiler_params=pltpu.CompilerParams(
            dimension_semantics=("parallel", "arbitrary")),
    )(keys_p, valid_p, qp, Wk, v_att, gvec, Wre, bre)


def _lang(tokens, embed, W, b):
    emb = embed[tokens]
    m = (tokens != 1).astype(jnp.float32)[..., None]
    pooled = (emb * m).sum(-2) / jnp.clip(m.sum(-2), 1.0, None)
    return jnp.tanh(pooled @ W + b)


def kernel(img_feat, img_loc, img_node1_id_list, img_node2_id_list, kg_entity,
           kg_e1_ids_list, kg_e2_ids_list, kg_edge, r_nodes, r_connects,
           r_type, W_img, b_img, W_loc, b_loc, W_rel, b_rel, embed, W_lang,
           b_lang, Wq, Wk, v_att, W_reason, b_reason):
    Wr1, Wr2 = W_rel[:D], W_rel[D:]
    x, node0, xa, y2 = _img_proj(
        img_feat.reshape(B * N, IMG), img_loc.reshape(B * N, LOC),
        W_img, b_img.reshape(1, D), W_loc, b_loc.reshape(1, D),
        Wr1, Wr2, b_rel.reshape(1, D))
    x = x.reshape(B, N, D)
    node0 = node0.reshape(B, N, D)
    y2 = y2.reshape(B, N, D)

    def _prep(idx, pad_val, offs):
        t = idx.reshape(B, 16, EPS).astype(jnp.int32) + offs
        pad = jnp.full((B, 16, NCH * CH - EPS), pad_val, jnp.int32)
        return jnp.concatenate([t, pad], axis=2).reshape(B, 16, NCH, CH)

    boffs = (jnp.arange(B, dtype=jnp.int32) * N)[:, None, None]
    i1p = _prep(img_node1_id_list, 0, boffs)
    i2p = _prep(img_node2_id_list, N, 0)
    pk = jnp.bitwise_or(jnp.left_shift(i2p, 15), i1p)
    acc = _edge_seg(xa, pk)
    cnt = _edge_cnt(pk).reshape(B, NP)[:, :N]
    node = node0 + acc[:, :N, :] + cnt[..., None] * y2

    kg_node = _lang(kg_entity, embed, W_lang, b_lang)
    kg_mask = jnp.sum(kg_entity == 1, axis=2) != TL
    kg_edge_f = _lang(kg_edge, embed, W_lang, b_lang)
    r_feat = _lang(r_nodes, embed, W_lang, b_lang)

    def kgmp(kn, e1, e2, ef):
        msg = kn[e1] + ef
        return kn + jnp.zeros((KG, D), jnp.float32).at[e2].add(msg)
    kg_node = jax.vmap(kgmp)(kg_node, kg_e1_ids_list, kg_e2_ids_list,
                             kg_edge_f)

    keys = jnp.concatenate([node, kg_node], axis=1)
    pad = KEYS_PAD - (N + KG)
    keys_p = jnp.pad(keys, ((0, 0), (0, pad), (0, 0)))
    valid = jnp.concatenate(
        [jnp.ones((B, N), jnp.float32), kg_mask.astype(jnp.float32),
         jnp.zeros((B, pad), jnp.float32)], axis=1)
    qp = r_feat @ Wq

    gate = (r_type[..., 0] != 0).astype(jnp.float32)
    conn_oh = jnp.sum(
        jax.nn.one_hot(r_connects, R, dtype=jnp.float32), axis=2)
    G = conn_oh + jax.vmap(jnp.diag)(gate)
    gvec = jnp.mean(G, axis=1, keepdims=True)

    out = _attention(keys_p, valid.reshape(B * (KEYS_PAD // KB), 1, KB), qp,
                     Wk, v_att.reshape(1, MID), gvec, W_reason,
                     b_reason.reshape(1, NC))
    return out.reshape(B, NC)


# speedup vs baseline: 1.0019x; 1.0019x over previous
import dataclasses
import functools

import jax
import jax.numpy as jnp
from jax import lax
from jax.experimental import pallas as pl
from jax.experimental.pallas import tpu as pltpu
from jax.experimental.pallas import tpu_sc as plsc

B, N, E = 2, 10000, 160000
IMG, LOC, D, MID = 1024, 5, 128, 128
KG, TL, KGE, R, CONN = 1000, 8, 4000, 8, 4
VOCAB, NC = 10000, 3000

AW = 128
NP = 10240
CH = 128
EPS = E // 16
NCH = 80
ZR = 640
CROW = NP // CH

BN = 2000
KB = 1376
KEYS_PAD = 11008


def _img_proj_body(feat_ref, loc_ref, Wi_ref, bi_ref, Wl_ref, bl_ref,
                   Wr1_ref, Wr2_ref, br_ref,
                   x_ref, node0_ref, y1_ref, y2_ref):
    x = jax.lax.dot_general(
        feat_ref[...], Wi_ref[...], (((1,), (0,)), ((), ())),
        preferred_element_type=jnp.float32) + bi_ref[...]
    locp = jax.lax.dot_general(
        loc_ref[...], Wl_ref[...], (((1,), (0,)), ((), ())),
        preferred_element_type=jnp.float32) + bl_ref[...]
    x_ref[...] = x
    node0_ref[...] = x + locp
    y1_ref[...] = jax.lax.dot_general(
        x, Wr1_ref[...], (((1,), (0,)), ((), ())),
        preferred_element_type=jnp.float32)
    y2_ref[...] = jax.lax.dot_general(
        x, Wr2_ref[...], (((1,), (0,)), ((), ())),
        preferred_element_type=jnp.float32) + br_ref[...]


def _img_proj(feat, loc, Wi, bi, Wl, bl, Wr1, Wr2, br):
    nb = (B * N) // BN
    grid = (nb,)
    row = lambda i: (i, 0)
    rep = lambda i: (0, 0)
    out_sd = jax.ShapeDtypeStruct((B * N, D), jnp.float32)
    out_aw = jax.ShapeDtypeStruct((B * N, AW), jnp.float32)
    return pl.pallas_call(
        _img_proj_body,
        grid=grid,
        in_specs=[
            pl.BlockSpec((BN, IMG), row),
            pl.BlockSpec((BN, LOC), row),
            pl.BlockSpec((IMG, D), rep),
            pl.BlockSpec((1, D), rep),
            pl.BlockSpec((LOC, D), rep),
            pl.BlockSpec((1, D), rep),
            pl.BlockSpec((D, D), rep),
            pl.BlockSpec((D, D), rep),
            pl.BlockSpec((1, D), rep),
        ],
        out_specs=[pl.BlockSpec((BN, D), row), pl.BlockSpec((BN, D), row),
                   pl.BlockSpec((BN, AW), row), pl.BlockSpec((BN, D), row)],
        out_shape=[out_sd, out_sd, out_aw, out_sd],
        compiler_params=pltpu.CompilerParams(
            dimension_semantics=("arbitrary",)),
    )(feat, loc, Wi, bi, Wl, bl, Wr1, Wr2, br)


def _sc_params():
    cp = pltpu.CompilerParams()
    if "needs_layout_passes" in pltpu.CompilerParams.__dataclass_fields__:
        cp = dataclasses.replace(cp, needs_layout_passes=False)
    return cp


def _cnt_body(pk_hbm, cnt_hbm, i2_v, zbuf, lcnt, iidx, cnt_sh):
    c = lax.axis_index("c")
    s = lax.axis_index("s")

    @pl.loop(0, CROW)
    def _(i):
        for g in range(CH // 16):
            lcnt[i, pl.ds(g * 16, 16)] = jnp.zeros((16,), jnp.float32)
            zbuf[0, pl.ds(g * 16, 16)] = jnp.zeros((16,), jnp.float32)

    for g in range(CROW // 16):
        iidx[pl.ds(g * 16, 16)] = lax.iota(jnp.int32, 16) + g * 16

    @pl.when(s < CROW // 8)
    def _():
        for t in range(8):
            pltpu.sync_copy(zbuf, cnt_sh.at[pl.ds(s * 8 + t, 1)])

    pltpu.sync_copy(pk_hbm.at[c, s], i2_v)
    plsc.subcore_barrier()

    @pl.loop(0, NCH)
    def _(j):
        for g in range(CH // 16):
            v = i2_v[j, pl.ds(g * 16, 16)]
            plsc.addupdate_scatter(
                lcnt, [lax.shift_right_logical(v, 22),
                       lax.bitwise_and(lax.shift_right_logical(v, 15), 127)],
                jnp.ones((16,), jnp.float32))

    pltpu.sync_copy(lcnt, cnt_sh.at[iidx], add=True)
    plsc.subcore_barrier()

    @pl.when(s < CROW // 8)
    def _():
        pltpu.sync_copy(cnt_sh.at[pl.ds(s * 8, 8)],
                        cnt_hbm.at[c].at[pl.ds(s * 8, 8)])


def _edge_cnt(pk):
    mesh = plsc.VectorSubcoreMesh(core_axis_name="c", subcore_axis_name="s")
    return pl.kernel(
        _cnt_body,
        out_type=jax.ShapeDtypeStruct((B, CROW, CH), jnp.float32),
        mesh=mesh,
        scratch_types=[
            pltpu.VMEM((NCH, CH), jnp.int32),
            pltpu.VMEM((1, CH), jnp.float32),
            pltpu.VMEM((CROW, CH), jnp.float32),
            pltpu.VMEM((CROW,), jnp.int32),
            pltpu.VMEM_SHARED((CROW, CH), jnp.float32),
        ],
        compiler_params=_sc_params(),
    )(pk)


def _edge_seg_body(xa_hbm, pk_hbm, out_hbm, i1_v, i2_v, bufa, acc_sh, sema):
    c = lax.axis_index("c")
    s = lax.axis_index("s")

    @pl.loop(0, CH)
    def _(i):
        for g in range(AW // 16):
            bufa[i, pl.ds(g * 16, 16)] = jnp.zeros((16,), jnp.float32)
    base = s * ZR
    for t in range(ZR // CH):
        pltpu.sync_copy(bufa, acc_sh.at[pl.ds(base + t * CH, CH)])

    pltpu.sync_copy(pk_hbm.at[c, s], i1_v)

    @pl.loop(0, NCH)
    def _(j):
        for g in range(CH // 16):
            v = i1_v[j, pl.ds(g * 16, 16)]
            i2_v[j, pl.ds(g * 16, 16)] = lax.shift_right_logical(v, 15)
            i1_v[j, pl.ds(g * 16, 16)] = lax.bitwise_and(v, 32767)
    plsc.subcore_barrier()

    @pl.loop(0, NCH)
    def _(j):
        pltpu.async_copy(xa_hbm.at[i1_v.at[j]], bufa, sema).wait()
        pltpu.sync_copy(bufa, acc_sh.at[i2_v.at[j]], add=True)

    plsc.subcore_barrier()
    pltpu.sync_copy(acc_sh.at[pl.ds(base, ZR)],
                    out_hbm.at[c].at[pl.ds(base, ZR)])


def _edge_seg(xa, pk):
    mesh = plsc.VectorSubcoreMesh(core_axis_name="c", subcore_axis_name="s")
    return pl.kernel(
        _edge_seg_body,
        out_type=jax.ShapeDtypeStruct((B, NP, AW), jnp.float32),
        mesh=mesh,
        scratch_types=[
            pltpu.VMEM((NCH, CH), jnp.int32),
            pltpu.VMEM((NCH, CH), jnp.int32),
            pltpu.VMEM((CH, AW), jnp.float32),
            pltpu.VMEM_SHARED((NP, AW), jnp.float32),
            pltpu.SemaphoreType.DMA,
        ],
        compiler_params=_sc_params(),
    )(xa, pk)


def _attn_body(keys_ref, valid_ref, qp_ref, Wk_ref, v_ref, gvec_ref,
               Wre_ref, bre_ref, out_ref, num_ref, den_ref):
    k = pl.program_id(1)
    nb = pl.num_programs(1)

    @pl.when(k == 0)
    def _():
        num_ref[...] = jnp.zeros_like(num_ref)
        den_ref[...] = jnp.zeros_like(den_ref)

    keys = keys_ref[0]
    kp = jax.lax.dot_general(
        keys, Wk_ref[...], (((1,), (0,)), ((), ())),
        preferred_element_type=jnp.float32)
    qp = qp_ref[0]
    valid = valid_ref[0]
    v = v_ref[...]
    for r in range(R):
        t = jnp.tanh(kp + qp[r:r + 1, :])
        s = jnp.sum(t * v, axis=1, keepdims=True)
        es = jnp.where(valid.T > 0.0, jnp.exp(s), 0.0)
        num_ref[r:r + 1, :] += jax.lax.dot_general(
            es, keys, (((0,), (0,)), ((), ())),
            preferred_element_type=jnp.float32)
        den_ref[r:r + 1, :] += jnp.broadcast_to(jnp.sum(es), (1, D))

    @pl.when(k == nb - 1)
    def _():
        node_re = num_ref[...] / den_ref[...]
        pooled = jax.lax.dot_general(
            gvec_ref[0], node_re, (((1,), (0,)), ((), ())),
            preferred_element_type=jnp.float32)
        out_ref[0] = jax.lax.dot_general(
            pooled, Wre_ref[...], (((1,), (0,)), ((), ())),
            preferred_element_type=jnp.float32) + bre_ref[...]


def _attention(keys_p, valid_p, qp, Wk, v_att, gvec, Wre, bre):
    nb = KEYS_PAD // KB
    grid = (B, nb)
    return pl.pallas_call(
        _attn_body,
        grid=grid,
        in_specs=[
            pl.BlockSpec((1, KB, D), lambda b, k: (b, k, 0)),
            pl.BlockSpec((1, 1, KB), lambda b, k: (b * nb + k, 0, 0)),
            pl.BlockSpec((1, R, MID), lambda b, k: (b, 0, 0)),
            pl.BlockSpec((D, MID), lambda b, k: (0, 0)),
            pl.BlockSpec((1, MID), lambda b, k: (0, 0)),
            pl.BlockSpec((1, 1, R), lambda b, k: (b, 0, 0)),
            pl.BlockSpec((D, NC), lambda b, k: (0, 0)),
            pl.BlockSpec((1, NC), lambda b, k: (0, 0)),
        ],
        out_specs=pl.BlockSpec((1, 1, NC), lambda b, k: (b, 0, 0)),
        out_shape=jax.ShapeDtypeStruct((B, 1, NC), jnp.float32),
        scratch_shapes=[
            pltpu.VMEM((R, D), jnp.float32),
            pltpu.VMEM((R, D), jnp.float32),
        ],
        compiler_params=pltpu.CompilerParams(
            dimension_semantics=("parallel", "arbitrary")),
    )(keys_p, valid_p, qp, Wk, v_att, gvec, Wre, bre)


def _lang(tokens, embed, W, b):
    emb = embed[tokens]
    m = (tokens != 1).astype(jnp.float32)[..., None]
    pooled = (emb * m).sum(-2) / jnp.clip(m.sum(-2), 1.0, None)
    return jnp.tanh(pooled @ W + b)


def kernel(img_feat, img_loc, img_node1_id_list, img_node2_id_list, kg_entity,
           kg_e1_ids_list, kg_e2_ids_list, kg_edge, r_nodes, r_connects,
           r_type, W_img, b_img, W_loc, b_loc, W_rel, b_rel, embed, W_lang,
           b_lang, Wq, Wk, v_att, W_reason, b_reason):
    Wr1, Wr2 = W_rel[:D], W_rel[D:]
    x, node0, xa, y2 = _img_proj(
        img_feat.reshape(B * N, IMG), img_loc.reshape(B * N, LOC),
        W_img, b_img.reshape(1, D), W_loc, b_loc.reshape(1, D),
        Wr1, Wr2, b_rel.reshape(1, D))
    x = x.reshape(B, N, D)
    node0 = node0.reshape(B, N, D)
    y2 = y2.reshape(B, N, D)

    def _prep(idx, pad_val, offs):
        t = idx.reshape(B, 16, EPS).astype(jnp.int32) + offs
        pad = jnp.full((B, 16, NCH * CH - EPS), pad_val, jnp.int32)
        return jnp.concatenate([t, pad], axis=2).reshape(B, 16, NCH, CH)

    boffs = (jnp.arange(B, dtype=jnp.int32) * N)[:, None, None]
    i1p = _prep(img_node1_id_list, 0, boffs)
    i2p = _prep(img_node2_id_list, N, 0)
    pk = jnp.bitwise_or(jnp.left_shift(i2p, 15), i1p)
    acc = _edge_seg(xa, pk)
    cnt = _edge_cnt(pk).reshape(B, NP)[:, :N]
    node = node0 + acc[:, :N, :] + cnt[..., None] * y2

    kg_node = _lang(kg_entity, embed, W_lang, b_lang)
    kg_mask = jnp.sum(kg_entity == 1, axis=2) != TL
    kg_edge_f = _lang(kg_edge, embed, W_lang, b_lang)
    r_feat = _lang(r_nodes, embed, W_lang, b_lang)

    def kgmp(kn, e1, e2, ef):
        msg = kn[e1] + ef
        return kn + jnp.zeros((KG, D), jnp.float32).at[e2].add(msg)
    kg_node = jax.vmap(kgmp)(kg_node, kg_e1_ids_list, kg_e2_ids_list,
                             kg_edge_f)

    keys = jnp.concatenate([node, kg_node], axis=1)
    pad = KEYS_PAD - (N + KG)
    keys_p = jnp.pad(keys, ((0, 0), (0, pad), (0, 0)))
    valid = jnp.concatenate(
        [jnp.ones((B, N), jnp.float32), kg_mask.astype(jnp.float32),
         jnp.zeros((B, pad), jnp.float32)], axis=1)
    qp = r_feat @ Wq

    gate = (r_type[..., 0] != 0).astype(jnp.float32)
    conn_oh = jnp.sum(
        jax.nn.one_hot(r_connects, R, dtype=jnp.float32), axis=2)
    G = conn_oh + jax.vmap(jnp.diag)(gate)
    gvec = jnp.mean(G, axis=1, keepdims=True)

    out = _attention(keys_p, valid.reshape(B * (KEYS_PAD // KB), 1, KB), qp,
                     Wk, v_att.reshape(1, MID), gvec, W_reason,
                     b_reason.reshape(1, NC))
    return out.reshape(B, NC)


# speedup vs baseline: 1.1490x; 1.1469x over previous
import dataclasses
import functools

import jax
import jax.numpy as jnp
from jax import lax
from jax.experimental import pallas as pl
from jax.experimental.pallas import tpu as pltpu
from jax.experimental.pallas import tpu_sc as plsc

B, N, E = 2, 10000, 160000
IMG, LOC, D, MID = 1024, 5, 128, 128
KG, TL, KGE, R, CONN = 1000, 8, 4000, 8, 4
VOCAB, NC = 10000, 3000

AW = 128
NP = 10240
CH = 128
EPS = E // 16
NCH = 80
ZR = 640
CROW = NP // CH

BN = 2000
KB = 1376
KEYS_PAD = 11008


def _img_proj_body(feat_ref, loc_ref, Wi_ref, bi_ref, Wl_ref, bl_ref,
                   Wr1_ref, Wr2_ref, br_ref,
                   x_ref, node0_ref, y1_ref, y2_ref):
    x = jax.lax.dot_general(
        feat_ref[...], Wi_ref[...], (((1,), (0,)), ((), ())),
        preferred_element_type=jnp.float32) + bi_ref[...]
    locp = jax.lax.dot_general(
        loc_ref[...], Wl_ref[...], (((1,), (0,)), ((), ())),
        preferred_element_type=jnp.float32) + bl_ref[...]
    x_ref[...] = x
    node0_ref[...] = x + locp
    y1_ref[...] = jax.lax.dot_general(
        x, Wr1_ref[...], (((1,), (0,)), ((), ())),
        preferred_element_type=jnp.float32)
    y2_ref[...] = jax.lax.dot_general(
        x, Wr2_ref[...], (((1,), (0,)), ((), ())),
        preferred_element_type=jnp.float32) + br_ref[...]


def _img_proj(feat, loc, Wi, bi, Wl, bl, Wr1, Wr2, br):
    nb = (B * N) // BN
    grid = (nb,)
    row = lambda i: (i, 0)
    rep = lambda i: (0, 0)
    out_sd = jax.ShapeDtypeStruct((B * N, D), jnp.float32)
    out_aw = jax.ShapeDtypeStruct((B * N, AW), jnp.float32)
    return pl.pallas_call(
        _img_proj_body,
        grid=grid,
        in_specs=[
            pl.BlockSpec((BN, IMG), row),
            pl.BlockSpec((BN, LOC), row),
            pl.BlockSpec((IMG, D), rep),
            pl.BlockSpec((1, D), rep),
            pl.BlockSpec((LOC, D), rep),
            pl.BlockSpec((1, D), rep),
            pl.BlockSpec((D, D), rep),
            pl.BlockSpec((D, D), rep),
            pl.BlockSpec((1, D), rep),
        ],
        out_specs=[pl.BlockSpec((BN, D), row), pl.BlockSpec((BN, D), row),
                   pl.BlockSpec((BN, AW), row), pl.BlockSpec((BN, D), row)],
        out_shape=[out_sd, out_sd, out_aw, out_sd],
        compiler_params=pltpu.CompilerParams(
            dimension_semantics=("arbitrary",)),
    )(feat, loc, Wi, bi, Wl, bl, Wr1, Wr2, br)


def _sc_params():
    cp = pltpu.CompilerParams()
    if "needs_layout_passes" in pltpu.CompilerParams.__dataclass_fields__:
        cp = dataclasses.replace(cp, needs_layout_passes=False)
    return cp


def _cnt_body(pk_hbm, cnt_hbm, i2_v, zbuf, lcnt, iidx, cnt_sh):
    c = lax.axis_index("c")
    s = lax.axis_index("s")

    @pl.loop(0, CROW)
    def _(i):
        for g in range(CH // 16):
            lcnt[i, pl.ds(g * 16, 16)] = jnp.zeros((16,), jnp.float32)
            zbuf[0, pl.ds(g * 16, 16)] = jnp.zeros((16,), jnp.float32)

    for g in range(CROW // 16):
        iidx[pl.ds(g * 16, 16)] = lax.iota(jnp.int32, 16) + g * 16

    @pl.when(s < CROW // 8)
    def _():
        for t in range(8):
            pltpu.sync_copy(zbuf, cnt_sh.at[pl.ds(s * 8 + t, 1)])

    pltpu.sync_copy(pk_hbm.at[c, s], i2_v)
    plsc.subcore_barrier()

    @pl.loop(0, NCH)
    def _(j):
        for g in range(CH // 16):
            v = i2_v[j, pl.ds(g * 16, 16)]
            plsc.addupdate_scatter(
                lcnt, [lax.shift_right_logical(v, 22),
                       lax.bitwise_and(lax.shift_right_logical(v, 15), 127)],
                jnp.ones((16,), jnp.float32))

    pltpu.sync_copy(lcnt, cnt_sh.at[iidx], add=True)
    plsc.subcore_barrier()

    @pl.when(s < CROW // 8)
    def _():
        pltpu.sync_copy(cnt_sh.at[pl.ds(s * 8, 8)],
                        cnt_hbm.at[c].at[pl.ds(s * 8, 8)])


def _edge_cnt(pk):
    mesh = plsc.VectorSubcoreMesh(core_axis_name="c", subcore_axis_name="s")
    return pl.kernel(
        _cnt_body,
        out_type=jax.ShapeDtypeStruct((B, CROW, CH), jnp.float32),
        mesh=mesh,
        scratch_types=[
            pltpu.VMEM((NCH, CH), jnp.int32),
            pltpu.VMEM((1, CH), jnp.float32),
            pltpu.VMEM((CROW, CH), jnp.float32),
            pltpu.VMEM((CROW,), jnp.int32),
            pltpu.VMEM_SHARED((CROW, CH), jnp.float32),
        ],
        compiler_params=_sc_params(),
    )(pk)


def _make_seg(nch, npad, zr):
    def body(xa_hbm, pk_hbm, out_hbm, pk_v, i1a, i2a, i1b, i2b,
             bufa, bufb, acc_sh, sema, semb):
        c = lax.axis_index("c")
        s = lax.axis_index("s")

        @pl.loop(0, CH)
        def _(i):
            for g in range(AW // 16):
                bufa[i, pl.ds(g * 16, 16)] = jnp.zeros((16,), jnp.float32)
        base = s * zr
        nfull, rem = divmod(zr, CH)
        for t in range(nfull):
            pltpu.sync_copy(bufa, acc_sh.at[pl.ds(base + t * CH, CH)])
        if rem:
            pltpu.sync_copy(bufa.at[pl.ds(0, rem)],
                            acc_sh.at[pl.ds(base + nfull * CH, rem)])

        pltpu.sync_copy(pk_hbm.at[c, s], pk_v)
        plsc.subcore_barrier()

        def _unpack(j, i1c, i2c):
            for g in range(CH // 16):
                v = pk_v[j, pl.ds(g * 16, 16)]
                i1c[0, pl.ds(g * 16, 16)] = lax.bitwise_and(v, 32767)
                i2c[0, pl.ds(g * 16, 16)] = lax.shift_right_logical(v, 15)

        _unpack(0, i1a, i2a)
        pltpu.make_async_copy(xa_hbm.at[i1a.at[0]], bufa, sema).start()

        @pl.loop(0, nch, step=2)
        def _(j):
            _unpack(j + 1, i1b, i2b)
            pltpu.make_async_copy(xa_hbm.at[i1b.at[0]], bufb, semb).start()
            pltpu.make_async_copy(xa_hbm.at[i1a.at[0]], bufa, sema).wait()
            pltpu.sync_copy(bufa, acc_sh.at[i2a.at[0]], add=True)

            @pl.when(j + 2 < nch)
            def _():
                _unpack(j + 2, i1a, i2a)
                pltpu.make_async_copy(xa_hbm.at[i1a.at[0]], bufa,
                                      sema).start()
            pltpu.make_async_copy(xa_hbm.at[i1b.at[0]], bufb, semb).wait()
            pltpu.sync_copy(bufb, acc_sh.at[i2b.at[0]], add=True)

        plsc.subcore_barrier()
        pltpu.sync_copy(acc_sh.at[pl.ds(base, zr)],
                        out_hbm.at[c].at[pl.ds(base, zr)])

    def call(xa, pk):
        mesh = plsc.VectorSubcoreMesh(core_axis_name="c",
                                      subcore_axis_name="s")
        return pl.kernel(
            body,
            out_type=jax.ShapeDtypeStruct((B, npad, AW), jnp.float32),
            mesh=mesh,
            scratch_types=[
                pltpu.VMEM((nch, CH), jnp.int32),
                pltpu.VMEM((1, CH), jnp.int32),
                pltpu.VMEM((1, CH), jnp.int32),
                pltpu.VMEM((1, CH), jnp.int32),
                pltpu.VMEM((1, CH), jnp.int32),
                pltpu.VMEM((CH, AW), jnp.float32),
                pltpu.VMEM((CH, AW), jnp.float32),
                pltpu.VMEM_SHARED((npad, AW), jnp.float32),
                pltpu.SemaphoreType.DMA,
                pltpu.SemaphoreType.DMA,
            ],
            compiler_params=_sc_params(),
        )(xa, pk)

    return call


_edge_seg = _make_seg(NCH, NP, ZR)

KGP = 1024
KNCH = 4
_kg_seg = _make_seg(KNCH, KGP, KGP // 16)


def _attn_body(keys_ref, valid_ref, qp_ref, Wk_ref, v_ref, gvec_ref,
               Wre_ref, bre_ref, out_ref, num_ref, den_ref):
    k = pl.program_id(1)
    nb = pl.num_programs(1)

    @pl.when(k == 0)
    def _():
        num_ref[...] = jnp.zeros_like(num_ref)
        den_ref[...] = jnp.zeros_like(den_ref)

    keys = keys_ref[0]
    kp = jax.lax.dot_general(
        keys, Wk_ref[...], (((1,), (0,)), ((), ())),
        preferred_element_type=jnp.float32)
    qp = qp_ref[0]
    valid = valid_ref[0]
    v = v_ref[...]
    for r in range(R):
        t = jnp.tanh(kp + qp[r:r + 1, :])
        s = jnp.sum(t * v, axis=1, keepdims=True)
        es = jnp.where(valid.T > 0.0, jnp.exp(s), 0.0)
        num_ref[r:r + 1, :] += jax.lax.dot_general(
            es, keys, (((0,), (0,)), ((), ())),
            preferred_element_type=jnp.float32)
        den_ref[r:r + 1, :] += jnp.broadcast_to(jnp.sum(es), (1, D))

    @pl.when(k == nb - 1)
    def _():
        node_re = num_ref[...] / den_ref[...]
        pooled = jax.lax.dot_general(
            gvec_ref[0], node_re, (((1,), (0,)), ((), ())),
            preferred_element_type=jnp.float32)
        out_ref[0] = jax.lax.dot_general(
            pooled, Wre_ref[...], (((1,), (0,)), ((), ())),
            preferred_element_type=jnp.float32) + bre_ref[...]


def _attention(keys_p, valid_p, qp, Wk, v_att, gvec, Wre, bre):
    nb = KEYS_PAD // KB
    grid = (B, nb)
    return pl.pallas_call(
        _attn_body,
        grid=grid,
        in_specs=[
            pl.BlockSpec((1, KB, D), lambda b, k: (b, k, 0)),
            pl.BlockSpec((1, 1, KB), lambda b, k: (b * nb + k, 0, 0)),
            pl.BlockSpec((1, R, MID), lambda b, k: (b, 0, 0)),
            pl.BlockSpec((D, MID), lambda b, k: (0, 0)),
            pl.BlockSpec((1, MID), lambda b, k: (0, 0)),
            pl.BlockSpec((1, 1, R), lambda b, k: (b, 0, 0)),
            pl.BlockSpec((D, NC), lambda b, k: (0, 0)),
            pl.BlockSpec((1, NC), lambda b, k: (0, 0)),
        ],
        out_specs=pl.BlockSpec((1, 1, NC), lambda b, k: (b, 0, 0)),
        out_shape=jax.ShapeDtypeStruct((B, 1, NC), jnp.float32),
        scratch_shapes=[
            pltpu.VMEM((R, D), jnp.float32),
            pltpu.VMEM((R, D), jnp.float32),
        ],
        compiler_params=pltpu.CompilerParams(
            dimension_semantics=("parallel", "arbitrary")),
    )(keys_p, valid_p, qp, Wk, v_att, gvec, Wre, bre)


def _lang(tokens, embed, W, b):
    emb = embed[tokens]
    m = (tokens != 1).astype(jnp.float32)[..., None]
    pooled = (emb * m).sum(-2) / jnp.clip(m.sum(-2), 1.0, None)
    return jnp.tanh(pooled @ W + b)


def kernel(img_feat, img_loc, img_node1_id_list, img_node2_id_list, kg_entity,
           kg_e1_ids_list, kg_e2_ids_list, kg_edge, r_nodes, r_connects,
           r_type, W_img, b_img, W_loc, b_loc, W_rel, b_rel, embed, W_lang,
           b_lang, Wq, Wk, v_att, W_reason, b_reason):
    Wr1, Wr2 = W_rel[:D], W_rel[D:]
    x, node0, xa, y2 = _img_proj(
        img_feat.reshape(B * N, IMG), img_loc.reshape(B * N, LOC),
        W_img, b_img.reshape(1, D), W_loc, b_loc.reshape(1, D),
        Wr1, Wr2, b_rel.reshape(1, D))
    x = x.reshape(B, N, D)
    node0 = node0.reshape(B, N, D)
    y2 = y2.reshape(B, N, D)

    def _prep(idx, pad_val, offs):
        t = idx.reshape(B, 16, EPS).astype(jnp.int32) + offs
        pad = jnp.full((B, 16, NCH * CH - EPS), pad_val, jnp.int32)
        return jnp.concatenate([t, pad], axis=2).reshape(B, 16, NCH, CH)

    boffs = (jnp.arange(B, dtype=jnp.int32) * N)[:, None, None]
    i1p = _prep(img_node1_id_list, 0, boffs)
    i2p = _prep(img_node2_id_list, N, 0)
    pk = jnp.bitwise_or(jnp.left_shift(i2p, 15), i1p)
    acc = _edge_seg(xa, pk)
    cnt = _edge_cnt(pk).reshape(B, NP)[:, :N]
    node = node0 + acc[:, :N, :] + cnt[..., None] * y2

    kg_node = _lang(kg_entity, embed, W_lang, b_lang)
    kg_mask = jnp.sum(kg_entity == 1, axis=2) != TL
    kg_edge_f = _lang(kg_edge, embed, W_lang, b_lang)
    r_feat = _lang(r_nodes, embed, W_lang, b_lang)

    T = jnp.concatenate([kg_node.reshape(B * KG, D),
                         kg_edge_f.reshape(B * KGE, D)], axis=0)
    koffs = (jnp.arange(B, dtype=jnp.int32) * KG)[:, None]
    src = jnp.concatenate(
        [kg_e1_ids_list.astype(jnp.int32) + koffs,
         (B * KG + jnp.arange(B * KGE, dtype=jnp.int32)).reshape(B, KGE)],
        axis=1)
    dst = jnp.concatenate([kg_e2_ids_list.astype(jnp.int32)] * 2, axis=1)
    kpk = jnp.bitwise_or(jnp.left_shift(dst, 15), src).reshape(B, 16, -1)
    kpk = jnp.concatenate(
        [kpk, jnp.full((B, 16, KNCH * CH - kpk.shape[2]), KG << 15,
                       jnp.int32)], axis=2).reshape(B, 16, KNCH, CH)
    kg_node = kg_node + _kg_seg(T, kpk)[:, :KG, :]

    keys = jnp.concatenate([node, kg_node], axis=1)
    pad = KEYS_PAD - (N + KG)
    keys_p = jnp.pad(keys, ((0, 0), (0, pad), (0, 0)))
    valid = jnp.concatenate(
        [jnp.ones((B, N), jnp.float32), kg_mask.astype(jnp.float32),
         jnp.zeros((B, pad), jnp.float32)], axis=1)
    qp = r_feat @ Wq

    gate = (r_type[..., 0] != 0).astype(jnp.float32)
    conn_oh = jnp.sum(
        jax.nn.one_hot(r_connects, R, dtype=jnp.float32), axis=2)
    G = conn_oh + jax.vmap(jnp.diag)(gate)
    gvec = jnp.mean(G, axis=1, keepdims=True)

    out = _attention(keys_p, valid.reshape(B * (KEYS_PAD // KB), 1, KB), qp,
                     Wk, v_att.reshape(1, MID), gvec, W_reason,
                     b_reason.reshape(1, NC))
    return out.reshape(B, NC)


# speedup vs baseline: 1.1513x; 1.0020x over previous
import dataclasses

import jax
import jax.numpy as jnp
from jax import lax
from jax.experimental import pallas as pl
from jax.experimental.pallas import tpu as pltpu
from jax.experimental.pallas import tpu_sc as plsc

B, N, E = 2, 10000, 160000
IMG, LOC, D, MID = 1024, 5, 128, 128
KG, TL, KGE, R, CONN = 1000, 8, 4000, 8, 4
VOCAB, NC = 10000, 3000

AW = 128
NP = 10240
CH = 128
EPS = E // 16
NCH = 80
ZR = 640
CROW = NP // CH

BN = 2000
KB = 1376
KEYS_PAD = 11008


def _img_proj_body(feat_ref, loc_ref, Wi_ref, bi_ref, Wl_ref, bl_ref,
                   Wr1_ref, Wr2_ref, br_ref,
                   x_ref, node0_ref, y1_ref, y2_ref):
    x = jax.lax.dot_general(
        feat_ref[...], Wi_ref[...], (((1,), (0,)), ((), ())),
        preferred_element_type=jnp.float32) + bi_ref[...]
    locp = jax.lax.dot_general(
        loc_ref[...], Wl_ref[...], (((1,), (0,)), ((), ())),
        preferred_element_type=jnp.float32) + bl_ref[...]
    x_ref[...] = x
    node0_ref[...] = x + locp
    y1_ref[...] = jax.lax.dot_general(
        x, Wr1_ref[...], (((1,), (0,)), ((), ())),
        preferred_element_type=jnp.float32)
    y2_ref[...] = jax.lax.dot_general(
        x, Wr2_ref[...], (((1,), (0,)), ((), ())),
        preferred_element_type=jnp.float32) + br_ref[...]


def _img_proj(feat, loc, Wi, bi, Wl, bl, Wr1, Wr2, br):
    nb = (B * N) // BN
    grid = (nb,)
    row = lambda i: (i, 0)
    rep = lambda i: (0, 0)
    out_sd = jax.ShapeDtypeStruct((B * N, D), jnp.float32)
    out_aw = jax.ShapeDtypeStruct((B * N, AW), jnp.float32)
    return pl.pallas_call(
        _img_proj_body,
        grid=grid,
        in_specs=[
            pl.BlockSpec((BN, IMG), row),
            pl.BlockSpec((BN, LOC), row),
            pl.BlockSpec((IMG, D), rep),
            pl.BlockSpec((1, D), rep),
            pl.BlockSpec((LOC, D), rep),
            pl.BlockSpec((1, D), rep),
            pl.BlockSpec((D, D), rep),
            pl.BlockSpec((D, D), rep),
            pl.BlockSpec((1, D), rep),
        ],
        out_specs=[pl.BlockSpec((BN, D), row), pl.BlockSpec((BN, D), row),
                   pl.BlockSpec((BN, AW), row), pl.BlockSpec((BN, D), row)],
        out_shape=[out_sd, out_sd, out_aw, out_sd],
        compiler_params=pltpu.CompilerParams(
            dimension_semantics=("arbitrary",)),
    )(feat, loc, Wi, bi, Wl, bl, Wr1, Wr2, br)


def _sc_params():
    cp = pltpu.CompilerParams()
    if "needs_layout_passes" in pltpu.CompilerParams.__dataclass_fields__:
        cp = dataclasses.replace(cp, needs_layout_passes=False)
    return cp


def _cnt_body(pk_hbm, cnt_hbm, i2_v, zbuf, lcnt, iidx, cnt_sh):
    c = lax.axis_index("c")
    s = lax.axis_index("s")

    @pl.loop(0, CROW)
    def _(i):
        for g in range(CH // 16):
            lcnt[i, pl.ds(g * 16, 16)] = jnp.zeros((16,), jnp.float32)
            zbuf[0, pl.ds(g * 16, 16)] = jnp.zeros((16,), jnp.float32)

    for g in range(CROW // 16):
        iidx[pl.ds(g * 16, 16)] = lax.iota(jnp.int32, 16) + g * 16

    @pl.when(s < CROW // 8)
    def _():
        for t in range(8):
            pltpu.sync_copy(zbuf, cnt_sh.at[pl.ds(s * 8 + t, 1)])

    pltpu.sync_copy(pk_hbm.at[c, s], i2_v)
    plsc.subcore_barrier()

    @pl.loop(0, NCH)
    def _(j):
        for g in range(CH // 16):
            v = i2_v[j, pl.ds(g * 16, 16)]
            plsc.addupdate_scatter(
                lcnt, [lax.shift_right_logical(v, 22),
                       lax.bitwise_and(lax.shift_right_logical(v, 15), 127)],
                jnp.ones((16,), jnp.float32))

    pltpu.sync_copy(lcnt, cnt_sh.at[iidx], add=True)
    plsc.subcore_barrier()

    @pl.when(s < CROW // 8)
    def _():
        pltpu.sync_copy(cnt_sh.at[pl.ds(s * 8, 8)],
                        cnt_hbm.at[c].at[pl.ds(s * 8, 8)])


def _edge_cnt(pk):
    mesh = plsc.VectorSubcoreMesh(core_axis_name="c", subcore_axis_name="s")
    return pl.kernel(
        _cnt_body,
        out_type=jax.ShapeDtypeStruct((B, CROW, CH), jnp.float32),
        mesh=mesh,
        scratch_types=[
            pltpu.VMEM((NCH, CH), jnp.int32),
            pltpu.VMEM((1, CH), jnp.float32),
            pltpu.VMEM((CROW, CH), jnp.float32),
            pltpu.VMEM((CROW,), jnp.int32),
            pltpu.VMEM_SHARED((CROW, CH), jnp.float32),
        ],
        compiler_params=_sc_params(),
    )(pk)


def _make_seg(nch, npad, zr):
    def body(xa_hbm, pk_hbm, out_hbm, pk_v, i1a, i2a, i1b, i2b,
             bufa, bufb, acc_sh, sema, semb):
        c = lax.axis_index("c")
        s = lax.axis_index("s")

        @pl.loop(0, CH)
        def _(i):
            for g in range(AW // 16):
                bufa[i, pl.ds(g * 16, 16)] = jnp.zeros((16,), jnp.float32)
        base = s * zr
        nfull, rem = divmod(zr, CH)
        for t in range(nfull):
            pltpu.sync_copy(bufa, acc_sh.at[pl.ds(base + t * CH, CH)])
        if rem:
            pltpu.sync_copy(bufa.at[pl.ds(0, rem)],
                            acc_sh.at[pl.ds(base + nfull * CH, rem)])

        pltpu.sync_copy(pk_hbm.at[c, s], pk_v)
        plsc.subcore_barrier()

        def _unpack(j, i1c, i2c):
            for g in range(CH // 16):
                v = pk_v[j, pl.ds(g * 16, 16)]
                i1c[0, pl.ds(g * 16, 16)] = lax.bitwise_and(v, 32767)
                i2c[0, pl.ds(g * 16, 16)] = lax.shift_right_logical(v, 15)

        _unpack(0, i1a, i2a)
        pltpu.make_async_copy(xa_hbm.at[i1a.at[0]], bufa, sema).start()

        @pl.loop(0, nch, step=2)
        def _(j):
            _unpack(j + 1, i1b, i2b)
            pltpu.make_async_copy(xa_hbm.at[i1b.at[0]], bufb, semb).start()
            pltpu.make_async_copy(xa_hbm.at[i1a.at[0]], bufa, sema).wait()
            pltpu.sync_copy(bufa, acc_sh.at[i2a.at[0]], add=True)

            @pl.when(j + 2 < nch)
            def _():
                _unpack(j + 2, i1a, i2a)
                pltpu.make_async_copy(xa_hbm.at[i1a.at[0]], bufa,
                                      sema).start()
            pltpu.make_async_copy(xa_hbm.at[i1b.at[0]], bufb, semb).wait()
            pltpu.sync_copy(bufb, acc_sh.at[i2b.at[0]], add=True)

        plsc.subcore_barrier()
        pltpu.sync_copy(acc_sh.at[pl.ds(base, zr)],
                        out_hbm.at[c].at[pl.ds(base, zr)])

    def call(xa, pk):
        mesh = plsc.VectorSubcoreMesh(core_axis_name="c",
                                      subcore_axis_name="s")
        return pl.kernel(
            body,
            out_type=jax.ShapeDtypeStruct((B, npad, AW), jnp.float32),
            mesh=mesh,
            scratch_types=[
                pltpu.VMEM((nch, CH), jnp.int32),
                pltpu.VMEM((1, CH), jnp.int32),
                pltpu.VMEM((1, CH), jnp.int32),
                pltpu.VMEM((1, CH), jnp.int32),
                pltpu.VMEM((1, CH), jnp.int32),
                pltpu.VMEM((CH, AW), jnp.float32),
                pltpu.VMEM((CH, AW), jnp.float32),
                pltpu.VMEM_SHARED((npad, AW), jnp.float32),
                pltpu.SemaphoreType.DMA,
                pltpu.SemaphoreType.DMA,
            ],
            compiler_params=_sc_params(),
        )(xa, pk)

    return call


_edge_seg = _make_seg(NCH, NP, ZR)

KGP = 1024
KNCH = 4
_kg_seg = _make_seg(KNCH, KGP, KGP // 16)


def _attn_body(keys_ref, valid_ref, qp_ref, Wk_ref, v_ref, gvec_ref,
               Wre_ref, bre_ref, out_ref, num_ref, den_ref):
    k = pl.program_id(1)
    nb = pl.num_programs(1)

    @pl.when(k == 0)
    def _():
        num_ref[...] = jnp.zeros_like(num_ref)
        den_ref[...] = jnp.zeros_like(den_ref)

    keys = keys_ref[0]
    kp = jax.lax.dot_general(
        keys, Wk_ref[...], (((1,), (0,)), ((), ())),
        preferred_element_type=jnp.float32)
    qp = qp_ref[0]
    valid = valid_ref[0]
    v = v_ref[...]
    for r in range(R):
        t = jnp.tanh(kp + qp[r:r + 1, :])
        s = jnp.sum(t * v, axis=1, keepdims=True)
        es = jnp.where(valid.T > 0.0, jnp.exp(s), 0.0)
        num_ref[r:r + 1, :] += jax.lax.dot_general(
            es, keys, (((0,), (0,)), ((), ())),
            preferred_element_type=jnp.float32)
        den_ref[r:r + 1, :] += jnp.broadcast_to(jnp.sum(es), (1, D))

    @pl.when(k == nb - 1)
    def _():
        node_re = num_ref[...] / den_ref[...]
        pooled = jax.lax.dot_general(
            gvec_ref[0], node_re, (((1,), (0,)), ((), ())),
            preferred_element_type=jnp.float32)
        out_ref[0] = jax.lax.dot_general(
            pooled, Wre_ref[...], (((1,), (0,)), ((), ())),
            preferred_element_type=jnp.float32) + bre_ref[...]


def _attention(keys_p, valid_p, qp, Wk, v_att, gvec, Wre, bre):
    nb = KEYS_PAD // KB
    grid = (B, nb)
    return pl.pallas_call(
        _attn_body,
        grid=grid,
        in_specs=[
            pl.BlockSpec((1, KB, D), lambda b, k: (b, k, 0)),
            pl.BlockSpec((1, 1, KB), lambda b, k: (b * nb + k, 0, 0)),
            pl.BlockSpec((1, R, MID), lambda b, k: (b, 0, 0)),
            pl.BlockSpec((D, MID), lambda b, k: (0, 0)),
            pl.BlockSpec((1, MID), lambda b, k: (0, 0)),
            pl.BlockSpec((1, 1, R), lambda b, k: (b, 0, 0)),
            pl.BlockSpec((D, NC), lambda b, k: (0, 0)),
            pl.BlockSpec((1, NC), lambda b, k: (0, 0)),
        ],
        out_specs=pl.BlockSpec((1, 1, NC), lambda b, k: (b, 0, 0)),
        out_shape=jax.ShapeDtypeStruct((B, 1, NC), jnp.float32),
        scratch_shapes=[
            pltpu.VMEM((R, D), jnp.float32),
            pltpu.VMEM((R, D), jnp.float32),
        ],
        compiler_params=pltpu.CompilerParams(
            dimension_semantics=("parallel", "arbitrary")),
    )(keys_p, valid_p, qp, Wk, v_att, gvec, Wre, bre)


def _lang(tokens, embed, W, b):
    emb = embed[tokens]
    m = (tokens != 1).astype(jnp.float32)[..., None]
    pooled = (emb * m).sum(-2) / jnp.clip(m.sum(-2), 1.0, None)
    return jnp.tanh(pooled @ W + b)


def kernel(img_feat, img_loc, img_node1_id_list, img_node2_id_list, kg_entity,
           kg_e1_ids_list, kg_e2_ids_list, kg_edge, r_nodes, r_connects,
           r_type, W_img, b_img, W_loc, b_loc, W_rel, b_rel, embed, W_lang,
           b_lang, Wq, Wk, v_att, W_reason, b_reason):
    Wr1, Wr2 = W_rel[:D], W_rel[D:]
    x, node0, xa, y2 = _img_proj(
        img_feat.reshape(B * N, IMG), img_loc.reshape(B * N, LOC),
        W_img, b_img.reshape(1, D), W_loc, b_loc.reshape(1, D),
        Wr1, Wr2, b_rel.reshape(1, D))
    x = x.reshape(B, N, D)
    node0 = node0.reshape(B, N, D)
    y2 = y2.reshape(B, N, D)

    def _prep(idx, pad_val, offs):
        t = idx.reshape(B, 16, EPS).astype(jnp.int32) + offs
        pad = jnp.full((B, 16, NCH * CH - EPS), pad_val, jnp.int32)
        return jnp.concatenate([t, pad], axis=2).reshape(B, 16, NCH, CH)

    boffs = (jnp.arange(B, dtype=jnp.int32) * N)[:, None, None]
    i1p = _prep(img_node1_id_list, 0, boffs)
    i2p = _prep(img_node2_id_list, N, 0)
    pk = jnp.bitwise_or(jnp.left_shift(i2p, 15), i1p)
    acc = _edge_seg(xa, pk)
    cnt = _edge_cnt(pk).reshape(B, NP)[:, :N]
    node = node0 + acc[:, :N, :] + cnt[..., None] * y2

    kg_node = _lang(kg_entity, embed, W_lang, b_lang)
    kg_mask = jnp.sum(kg_entity == 1, axis=2) != TL
    kg_edge_f = _lang(kg_edge, embed, W_lang, b_lang)
    r_feat = _lang(r_nodes, embed, W_lang, b_lang)

    T = jnp.concatenate([kg_node.reshape(B * KG, D),
                         kg_edge_f.reshape(B * KGE, D)], axis=0)
    koffs = (jnp.arange(B, dtype=jnp.int32) * KG)[:, None]
    src = jnp.concatenate(
        [kg_e1_ids_list.astype(jnp.int32) + koffs,
         (B * KG + jnp.arange(B * KGE, dtype=jnp.int32)).reshape(B, KGE)],
        axis=1)
    dst = jnp.concatenate([kg_e2_ids_list.astype(jnp.int32)] * 2, axis=1)
    kpk = jnp.bitwise_or(jnp.left_shift(dst, 15), src).reshape(B, 16, -1)
    kpk = jnp.concatenate(
        [kpk, jnp.full((B, 16, KNCH * CH - kpk.shape[2]), KG << 15,
                       jnp.int32)], axis=2).reshape(B, 16, KNCH, CH)
    kg_node = kg_node + _kg_seg(T, kpk)[:, :KG, :]

    keys = jnp.concatenate([node, kg_node], axis=1)
    pad = KEYS_PAD - (N + KG)
    keys_p = jnp.pad(keys, ((0, 0), (0, pad), (0, 0)))
    valid = jnp.concatenate(
        [jnp.ones((B, N), jnp.float32), kg_mask.astype(jnp.float32),
         jnp.zeros((B, pad), jnp.float32)], axis=1)
    qp = r_feat @ Wq

    gate = (r_type[..., 0] != 0).astype(jnp.float32)
    conn_oh = jnp.sum(
        jax.nn.one_hot(r_connects, R, dtype=jnp.float32), axis=2)
    G = conn_oh + jax.vmap(jnp.diag)(gate)
    gvec = jnp.mean(G, axis=1, keepdims=True)

    out = _attention(keys_p, valid.reshape(B * (KEYS_PAD // KB), 1, KB), qp,
                     Wk, v_att.reshape(1, MID), gvec, W_reason,
                     b_reason.reshape(1, NC))
    return out.reshape(B, NC)
